# baseline (speedup 1.0000x reference)
import functools

import jax
import jax.numpy as jnp
from jax import lax
from jax.experimental import pallas as pl
from jax.experimental.pallas import tpu as pltpu
from jax.experimental.pallas import tpu_sc as plsc

_D = 16
_NC = 2
_NS = 16
_NW = _NC * _NS


def _sc_gather(w, idx_flat):
    n_idx = idx_flat.shape[0]
    b_per_w = n_idx // _NW
    mesh = plsc.VectorSubcoreMesh(core_axis_name="c", subcore_axis_name="s")

    @functools.partial(
        pl.kernel,
        mesh=mesh,
        out_type=jax.ShapeDtypeStruct((n_idx, _D), jnp.float32),
        compiler_params=pltpu.CompilerParams(use_tc_tiling_on_sc=False),
        scratch_types=[
            pltpu.VMEM((b_per_w,), jnp.int32),
            pltpu.VMEM((b_per_w, _D), jnp.float32),
            pltpu.SemaphoreType.DMA,
        ],
    )
    def gather_k(w_hbm, idx_hbm, out_hbm, idx_v, rows_v, sem):
        wid = lax.axis_index("s") * _NC + lax.axis_index("c")
        base = wid * b_per_w
        pltpu.sync_copy(idx_hbm.at[pl.ds(base, b_per_w)], idx_v)
        pltpu.async_copy(w_hbm.at[idx_v], rows_v, sem).wait()
        pltpu.sync_copy(rows_v, out_hbm.at[pl.ds(base, b_per_w)])

    return gather_k(w, idx_flat)


def _hdist_body(x_ref, s_ref, o_ref):
    x = x_ref[...]
    u = x[:, :_D]
    v = x[:, _D:]
    su = jnp.sum(u * u, axis=1)
    sv = jnp.sum(v * v, axis=1)
    d = u - v
    z = 2.0 * jnp.sum(d * d, axis=1)
    uu = 1.0 + z / ((1.0 - su) * (1.0 - sv))
    acosh = jnp.log(uu + jnp.sqrt(uu * uu - 1.0))
    o_ref[...] = acosh / (1.0 + s_ref[0])


def _tc_math(g2, scale, blk):
    b = g2.shape[0]
    return pl.pallas_call(
        _hdist_body,
        grid=(b // blk,),
        in_specs=[
            pl.BlockSpec((blk, 2 * _D), lambda i: (i, 0)),
            pl.BlockSpec(memory_space=pltpu.SMEM),
        ],
        out_specs=pl.BlockSpec((blk,), lambda i: (i,)),
        out_shape=jax.ShapeDtypeStruct((b,), jnp.float32),
    )(g2, scale)


def kernel(idx, w, scale):
    b = idx.shape[0]
    idx_flat = idx.reshape(-1).astype(jnp.int32)
    g = _sc_gather(w, idx_flat)
    g2 = g.reshape(b, 2 * _D)
    return _tc_math(g2, scale, blk=2048)
